# Initial kernel scaffold; baseline (speedup 1.0000x reference)
import jax
import jax.numpy as jnp
from jax.experimental import pallas as pl


def kernel(time_seqs, hour_table, minute_table, second_table):
    raise NotImplementedError("write your pallas kernel here")


# speedup vs baseline: 13.0473x; 13.0473x over previous
import jax
import jax.numpy as jnp
from jax import lax
from jax.experimental import pallas as pl
from jax.experimental.pallas import tpu as pltpu
from jax.experimental.pallas import tpu_sc as plsc

NC = 2
NS = 16
NW = NC * NS
L = 16

B, S = 16384, 200
N = B * S
PER_W = N // NW
C = 2048
CHUNKS = PER_W // C
GROUPS = C // L
OUT_W = C * 24

MIN_OFF = 24 * 8
SEC_OFF = 84 * 8
ZERO_OFF = 144 * 8
TBL_W = ZERO_OFF + L


def _body(ts_hbm, tbl_hbm, out_hbm, tbl_v, in_v, out_v,
          in_sem0, in_sem1, out_sem0, out_sem1):
    pltpu.sync_copy(tbl_hbm, tbl_v)
    lanes = lax.iota(jnp.int32, L)
    lanes24 = lanes * 24

    wid = lax.axis_index("s") * NC + lax.axis_index("c")
    w_base = wid * PER_W

    zoff_v = jnp.full((L,), ZERO_OFF, jnp.int32)
    in_sems = (in_sem0, in_sem1)
    out_sems = (out_sem0, out_sem1)

    def in_copy(ci, s):
        src = ts_hbm.at[pl.ds(pl.multiple_of(w_base + ci * C, C), C)]
        return pltpu.make_async_copy(src, in_v.at[pl.ds(s * C, C)], in_sems[s])

    def out_copy(ci, s):
        dst = out_hbm.at[pl.ds(pl.multiple_of((w_base + ci * C) * 24, OUT_W), OUT_W)]
        return pltpu.make_async_copy(out_v.at[pl.ds(s * OUT_W, OUT_W)], dst,
                                     out_sems[s])

    in_copy(0, 0).start()

    def chunk_work(ci, s):
        in_copy(ci, s).wait()

        @pl.when(ci + 1 < CHUNKS)
        def _():
            in_copy(ci + 1, 1 - s).start()

        @pl.when(ci >= 2)
        def _():
            out_copy(ci - 2, s).wait()

        def group_body(g, _):
            v = in_v[pl.ds(s * C + g * L, L)]
            t = jnp.maximum(v, 0)
            day = jnp.right_shift(jnp.right_shift(t, 7) * 6214, 22)
            tod = t - day * 86400
            hour = jnp.right_shift(jnp.right_shift(tod, 4) * 4661, 20)
            r = tod - hour * 3600
            minute = jnp.right_shift(r * 34953, 21)
            second = r - minute * 60
            valid = v > 0
            zh = jnp.where(valid, hour * 8, zoff_v)
            zm = jnp.where(valid, MIN_OFF + minute * 8, zoff_v)
            zs = jnp.where(valid, SEC_OFF + second * 8, zoff_v)
            obase = s * OUT_W + g * (L * 24) + lanes24
            for d in range(24):
                band = zh if d < 8 else (zm if d < 16 else zs)
                vals = plsc.load_gather(tbl_v, [band + (d % 8) if d % 8 else band])
                plsc.store_scatter(out_v, [obase + d if d else obase], vals)
            return 0

        lax.fori_loop(0, GROUPS, group_body, 0)
        out_copy(ci, s).start()

    def chunk_body(ci, _):
        parity = jnp.bitwise_and(ci, 1)

        @pl.when(parity == 0)
        def _():
            chunk_work(ci, 0)

        @pl.when(parity == 1)
        def _():
            chunk_work(ci, 1)

        return 0

    lax.fori_loop(0, CHUNKS, chunk_body, 0)
    out_copy(CHUNKS - 2, 0).wait()
    out_copy(CHUNKS - 1, 1).wait()


@jax.jit
def _encode(ts_flat, tbl_flat):
    mesh = plsc.VectorSubcoreMesh(
        core_axis_name="c", subcore_axis_name="s",
        num_cores=NC, num_subcores=NS)
    return pl.kernel(
        _body,
        out_type=jax.ShapeDtypeStruct((N * 24,), jnp.float32),
        mesh=mesh,
        compiler_params=pltpu.CompilerParams(needs_layout_passes=False),
        scratch_types=[
            pltpu.VMEM((TBL_W,), jnp.float32),
            pltpu.VMEM((2 * C,), jnp.int32),
            pltpu.VMEM((2 * OUT_W,), jnp.float32),
            pltpu.SemaphoreType.DMA,
            pltpu.SemaphoreType.DMA,
            pltpu.SemaphoreType.DMA,
            pltpu.SemaphoreType.DMA,
        ],
    )(ts_flat, tbl_flat)


def kernel(time_seqs, hour_table, minute_table, second_table):
    ts_flat = time_seqs.reshape(-1).astype(jnp.int32)
    tbl_flat = jnp.concatenate([
        hour_table.reshape(-1).astype(jnp.float32),
        minute_table.reshape(-1).astype(jnp.float32),
        second_table.reshape(-1).astype(jnp.float32),
        jnp.zeros((L,), jnp.float32),
    ])
    out = _encode(ts_flat, tbl_flat)
    return out.reshape(B, S, 24)


# speedup vs baseline: 16.9167x; 1.2966x over previous
import jax
import jax.numpy as jnp
from jax import lax
from jax.experimental import pallas as pl
from jax.experimental.pallas import tpu as pltpu
from jax.experimental.pallas import tpu_sc as plsc

NC = 2
NS = 16
NW = NC * NS
L = 16

B, S = 16384, 200
N = B * S
PER_W = N // NW
C = 2048
CHUNKS = PER_W // C
GROUPS = C // L
OUT_W = C * 24

MIN_OFF = 24 * 8
SEC_OFF = 84 * 8
ZERO_OFF = 144 * 8
TBL_W = ZERO_OFF + L


def _body(ts_hbm, tbl_hbm, out_hbm, tbl_v, in_v, out_v,
          in_sem0, in_sem1, out_sem0, out_sem1):
    pltpu.sync_copy(tbl_hbm, tbl_v)
    lanes = lax.iota(jnp.int32, L)
    lanes24 = lanes * 24

    wid = lax.axis_index("s") * NC + lax.axis_index("c")
    w_base = wid * PER_W

    zoff_v = jnp.full((L,), ZERO_OFF, jnp.int32)
    in_sems = (in_sem0, in_sem1)
    out_sems = (out_sem0, out_sem1)

    def in_copy(ci, s):
        src = ts_hbm.at[pl.ds(pl.multiple_of(w_base + ci * C, C), C)]
        return pltpu.make_async_copy(src, in_v.at[pl.ds(s * C, C)], in_sems[s])

    def out_copy(ci, s):
        dst = out_hbm.at[pl.ds(pl.multiple_of((w_base + ci * C) * 24, OUT_W), OUT_W)]
        return pltpu.make_async_copy(out_v.at[pl.ds(s * OUT_W, OUT_W)], dst,
                                     out_sems[s])

    in_copy(0, 0).start()

    def chunk_work(ci, s):
        in_copy(ci, s).wait()

        @pl.when(ci + 1 < CHUNKS)
        def _():
            in_copy(ci + 1, 1 - s).start()

        @pl.when(ci >= 2)
        def _():
            out_copy(ci - 2, s).wait()

        @plsc.parallel_loop(0, GROUPS, unroll=4)
        def group_body(g):
            v = in_v[pl.ds(s * C + g * L, L)]
            t = jnp.maximum(v, 0)
            day = jnp.right_shift(jnp.right_shift(t, 7) * 6214, 22)
            tod = t - day * 86400
            hour = jnp.right_shift(jnp.right_shift(tod, 4) * 4661, 20)
            r = tod - hour * 3600
            minute = jnp.right_shift(r * 34953, 21)
            second = r - minute * 60
            valid = v > 0
            zh = jnp.where(valid, hour * 8, zoff_v)
            zm = jnp.where(valid, MIN_OFF + minute * 8, zoff_v)
            zs = jnp.where(valid, SEC_OFF + second * 8, zoff_v)
            obase = s * OUT_W + g * (L * 24) + lanes24
            for d in range(24):
                band = zh if d < 8 else (zm if d < 16 else zs)
                vals = plsc.load_gather(tbl_v, [band + (d % 8) if d % 8 else band])
                plsc.store_scatter(out_v, [obase + d if d else obase], vals)

        out_copy(ci, s).start()

    def chunk_body(ci, _):
        parity = jnp.bitwise_and(ci, 1)

        @pl.when(parity == 0)
        def _():
            chunk_work(ci, 0)

        @pl.when(parity == 1)
        def _():
            chunk_work(ci, 1)

        return 0

    lax.fori_loop(0, CHUNKS, chunk_body, 0)
    out_copy(CHUNKS - 2, 0).wait()
    out_copy(CHUNKS - 1, 1).wait()


@jax.jit
def _encode(ts_flat, tbl_flat):
    mesh = plsc.VectorSubcoreMesh(
        core_axis_name="c", subcore_axis_name="s",
        num_cores=NC, num_subcores=NS)
    return pl.kernel(
        _body,
        out_type=jax.ShapeDtypeStruct((N * 24,), jnp.float32),
        mesh=mesh,
        compiler_params=pltpu.CompilerParams(
            needs_layout_passes=False, disable_bounds_checks=True),
        scratch_types=[
            pltpu.VMEM((TBL_W,), jnp.float32),
            pltpu.VMEM((2 * C,), jnp.int32),
            pltpu.VMEM((2 * OUT_W,), jnp.float32),
            pltpu.SemaphoreType.DMA,
            pltpu.SemaphoreType.DMA,
            pltpu.SemaphoreType.DMA,
            pltpu.SemaphoreType.DMA,
        ],
    )(ts_flat, tbl_flat)


def kernel(time_seqs, hour_table, minute_table, second_table):
    ts_flat = time_seqs.reshape(-1).astype(jnp.int32)
    tbl_flat = jnp.concatenate([
        hour_table.reshape(-1).astype(jnp.float32),
        minute_table.reshape(-1).astype(jnp.float32),
        second_table.reshape(-1).astype(jnp.float32),
        jnp.zeros((L,), jnp.float32),
    ])
    out = _encode(ts_flat, tbl_flat)
    return out.reshape(B, S, 24)
